# Initial kernel scaffold; baseline (speedup 1.0000x reference)
import jax
import jax.numpy as jnp
from jax.experimental import pallas as pl


def kernel(x, edge_index, batch, W_rel, b_rel, W_root, gamma, beta):
    raise NotImplementedError("write your pallas kernel here")


# speedup vs baseline: 5.4686x; 5.4686x over previous
import functools

import jax
import jax.numpy as jnp
from jax import lax
from jax.experimental import pallas as pl
from jax.experimental.pallas import tpu as pltpu
from jax.experimental.pallas import tpu_sc as plsc

N_NODES = 10000
N_EDGES = 320000
D = 128
EPS = 1e-5

NC = 2
NS = 16
NW = NC * NS
E_PER_TILE = N_EDGES // NW
K = 80
NCHUNK = E_PER_TILE // K
ROWS_MAIN = 624
ROWS_TAIL_OFF = ROWS_MAIN * NS
ROWS_TAIL = N_NODES - ROWS_TAIL_OFF


def _sc_aggregate(src_hbm, dst_hbm, x_hbm, zeros_hbm, out_hbm,
                  src_v, dst_v, rows_v, aggr_sh, sem):
    c = lax.axis_index("c")
    s = lax.axis_index("s")
    wid = s * NC + c
    base_e = wid * E_PER_TILE

    r0 = s * ROWS_MAIN
    pltpu.sync_copy(zeros_hbm.at[pl.ds(r0, ROWS_MAIN)],
                    aggr_sh.at[pl.ds(r0, ROWS_MAIN)])

    @pl.when(s == NS - 1)
    def _():
        pltpu.sync_copy(zeros_hbm.at[pl.ds(ROWS_TAIL_OFF, ROWS_TAIL)],
                        aggr_sh.at[pl.ds(ROWS_TAIL_OFF, ROWS_TAIL)])

    plsc.subcore_barrier()

    def body(i, carry):
        off = base_e + i * K
        pltpu.sync_copy(src_hbm.at[pl.ds(off, K)], src_v)
        pltpu.sync_copy(dst_hbm.at[pl.ds(off, K)], dst_v)
        pltpu.async_copy(x_hbm.at[src_v], rows_v, sem).wait()
        pltpu.sync_copy(rows_v, aggr_sh.at[dst_v], add=True)
        return carry

    lax.fori_loop(0, NCHUNK, body, 0)
    plsc.subcore_barrier()

    pltpu.sync_copy(aggr_sh.at[pl.ds(r0, ROWS_MAIN)],
                    out_hbm.at[c, pl.ds(r0, ROWS_MAIN)])

    @pl.when(s == NS - 1)
    def _():
        pltpu.sync_copy(aggr_sh.at[pl.ds(ROWS_TAIL_OFF, ROWS_TAIL)],
                        out_hbm.at[c, pl.ds(ROWS_TAIL_OFF, ROWS_TAIL)])


_sc_aggregate_call = pl.kernel(
    _sc_aggregate,
    out_type=jax.ShapeDtypeStruct((NC, N_NODES, D), jnp.float32),
    mesh=plsc.VectorSubcoreMesh(core_axis_name="c", subcore_axis_name="s",
                                num_cores=NC, num_subcores=NS),
    scratch_types=[
        pltpu.VMEM((K,), jnp.int32),
        pltpu.VMEM((K,), jnp.int32),
        pltpu.VMEM((K, D), jnp.float32),
        pltpu.VMEM_SHARED((N_NODES, D), jnp.float32),
        pltpu.SemaphoreType.DMA,
    ],
)


def _tc_finish(parts_ref, x_ref, wrT_ref, b_ref, wrootT_ref,
               gamma_ref, beta_ref, out_ref):
    aggr = parts_ref[0] + parts_ref[1]
    h = (jnp.dot(aggr, wrT_ref[...], preferred_element_type=jnp.float32)
         + jnp.dot(x_ref[...], wrootT_ref[...],
                   preferred_element_type=jnp.float32)
         + b_ref[...])
    mean = jnp.mean(h, axis=0, keepdims=True)
    d = h - mean
    var = jnp.mean(d * d, axis=0, keepdims=True)
    inv = lax.rsqrt(var + EPS)
    out_ref[...] = jnp.maximum(d * inv * gamma_ref[...] + beta_ref[...], 0.0)


_tc_finish_call = pl.pallas_call(
    _tc_finish,
    out_shape=jax.ShapeDtypeStruct((N_NODES, D), jnp.float32),
)


@jax.jit
def kernel(x, edge_index, batch, W_rel, b_rel, W_root, gamma, beta):
    src = edge_index[0].astype(jnp.int32)
    dst = edge_index[1].astype(jnp.int32)
    zeros = jnp.zeros((N_NODES, D), jnp.float32)
    parts = _sc_aggregate_call(src, dst, x, zeros)
    return _tc_finish_call(parts, x, W_rel.T, b_rel.reshape(1, D),
                           W_root.T, gamma.reshape(1, D), beta.reshape(1, D))


# speedup vs baseline: 11.7659x; 2.1515x over previous
import functools

import jax
import jax.numpy as jnp
from jax import lax
from jax.experimental import pallas as pl
from jax.experimental.pallas import tpu as pltpu
from jax.experimental.pallas import tpu_sc as plsc

N_NODES = 10000
N_EDGES = 320000
D = 128
EPS = 1e-5

NC = 2
NS = 16
NW = NC * NS
E_PER_TILE = N_EDGES // NW
K = 80
NCHUNK = E_PER_TILE // K
NI = 4
NG = 2
U = 4
ROWS_MAIN = 624
ROWS_TAIL_OFF = ROWS_MAIN * NS
ROWS_TAIL = N_NODES - ROWS_TAIL_OFF


def _sc_aggregate(src_hbm, dst_hbm, x_hbm, zeros_hbm, out_hbm,
                  iv0, iv1, iv2, iv3, dv0, dv1, dv2, dv3,
                  rows0, rows1, aggr_sh,
                  is0, is1, is2, is3, ds0, ds1, ds2, ds3, gs0, gs1):
    ivs = (iv0, iv1, iv2, iv3)
    dvs = (dv0, dv1, dv2, dv3)
    isems = (is0, is1, is2, is3)
    dsems = (ds0, ds1, ds2, ds3)
    rows = (rows0, rows1)
    gsems = (gs0, gs1)

    c = lax.axis_index("c")
    s = lax.axis_index("s")
    wid = s * NC + c

    r0 = s * ROWS_MAIN
    pltpu.sync_copy(zeros_hbm.at[pl.ds(r0, ROWS_MAIN)],
                    aggr_sh.at[pl.ds(r0, ROWS_MAIN)])

    @pl.when(s == NS - 1)
    def _():
        pltpu.sync_copy(zeros_hbm.at[pl.ds(ROWS_TAIL_OFF, ROWS_TAIL)],
                        aggr_sh.at[pl.ds(ROWS_TAIL_OFF, ROWS_TAIL)])

    cbase = wid * NCHUNK

    def fire_idx(i, p):
        pltpu.async_copy(src_hbm.at[cbase + i], ivs[p], isems[p])
        pltpu.async_copy(dst_hbm.at[cbase + i], dvs[p], dsems[p])

    def fire_gather(i_unused, p):
        pltpu.make_async_copy(src_hbm.at[0], ivs[p], isems[p]).wait()
        pltpu.async_copy(x_hbm.at[ivs[p]], rows[p % NG], gsems[p % NG])

    def scatter(i_unused, p):
        pltpu.make_async_copy(x_hbm.at[pl.ds(0, K)], rows[p % NG],
                              gsems[p % NG]).wait()
        pltpu.make_async_copy(dst_hbm.at[0], dvs[p], dsems[p]).wait()
        pltpu.sync_copy(rows[p % NG], aggr_sh.at[dvs[p]], add=True)

    for b in range(NI):
        fire_idx(b, b)
    fire_gather(0, 0)

    plsc.subcore_barrier()

    def body(j, carry):
        i0 = j * U
        for b in range(U):
            i = i0 + b
            fire_gather(i + 1, (b + 1) % NI)
            scatter(i, b)
            fire_idx(i + NI, b)
        return carry

    n_main = (NCHUNK - NI) // U * U
    lax.fori_loop(0, n_main // U, body, 0)

    for i in range(n_main, NCHUNK):
        if i + 1 < NCHUNK:
            fire_gather(i + 1, (i + 1) % NI)
        scatter(i, i % NI)
        if i + NI < NCHUNK:
            fire_idx(i + NI, i % NI)

    plsc.subcore_barrier()

    pltpu.sync_copy(aggr_sh.at[pl.ds(r0, ROWS_MAIN)],
                    out_hbm.at[c, pl.ds(r0, ROWS_MAIN)])

    @pl.when(s == NS - 1)
    def _():
        pltpu.sync_copy(aggr_sh.at[pl.ds(ROWS_TAIL_OFF, ROWS_TAIL)],
                        out_hbm.at[c, pl.ds(ROWS_TAIL_OFF, ROWS_TAIL)])


_sc_aggregate_call = pl.kernel(
    _sc_aggregate,
    out_type=jax.ShapeDtypeStruct((NC, N_NODES, D), jnp.float32),
    mesh=plsc.VectorSubcoreMesh(core_axis_name="c", subcore_axis_name="s",
                                num_cores=NC, num_subcores=NS),
    scratch_types=(
        [pltpu.VMEM((K,), jnp.int32) for _ in range(NI)]
        + [pltpu.VMEM((K,), jnp.int32) for _ in range(NI)]
        + [pltpu.VMEM((K, D), jnp.float32) for _ in range(NG)]
        + [pltpu.VMEM_SHARED((N_NODES, D), jnp.float32)]
        + [pltpu.SemaphoreType.DMA for _ in range(2 * NI + NG)]
    ),
)


def _tc_finish(parts_ref, x_ref, wrT_ref, b_ref, wrootT_ref,
               gamma_ref, beta_ref, out_ref):
    aggr = parts_ref[0] + parts_ref[1]
    h = (jnp.dot(aggr, wrT_ref[...], preferred_element_type=jnp.float32)
         + jnp.dot(x_ref[...], wrootT_ref[...],
                   preferred_element_type=jnp.float32)
         + b_ref[...])
    mean = jnp.mean(h, axis=0, keepdims=True)
    d = h - mean
    var = jnp.mean(d * d, axis=0, keepdims=True)
    inv = lax.rsqrt(var + EPS)
    out_ref[...] = jnp.maximum(d * inv * gamma_ref[...] + beta_ref[...], 0.0)


_tc_finish_call = pl.pallas_call(
    _tc_finish,
    out_shape=jax.ShapeDtypeStruct((N_NODES, D), jnp.float32),
)


@jax.jit
def kernel(x, edge_index, batch, W_rel, b_rel, W_root, gamma, beta):
    src = edge_index[0].astype(jnp.int32).reshape(NW * NCHUNK, K)
    dst = edge_index[1].astype(jnp.int32).reshape(NW * NCHUNK, K)
    zeros = jnp.zeros((N_NODES, D), jnp.float32)
    parts = _sc_aggregate_call(src, dst, x, zeros)
    return _tc_finish_call(parts, x, W_rel.T, b_rel.reshape(1, D),
                           W_root.T, gamma.reshape(1, D), beta.reshape(1, D))
